# baseline (speedup 1.0000x reference)
import jax
import jax.numpy as jnp
from jax.experimental import pallas as pl
from jax.experimental.pallas import tpu as pltpu

B, C, H, W = 8, 320, 48, 48
HW = H * W


CB = 64


def _scale_body(n_ref, g_ref, x_ref, o_ref):
    g = jnp.abs(g_ref[0, 0, 0])
    o_ref[...] = g[None, :, None, None] * x_ref[...]


def kernel(x, n, gain_matrix):
    g3 = gain_matrix.reshape(B, C // CB, 1, CB)
    out = pl.pallas_call(
        _scale_body,
        grid_spec=pltpu.PrefetchScalarGridSpec(
            num_scalar_prefetch=1,
            grid=(B, C // CB),
            in_specs=[
                pl.BlockSpec((1, 1, 1, CB), lambda b, c, n_ref: (n_ref[b], c, 0, 0)),
                pl.BlockSpec((1, CB, H, W), lambda b, c, n_ref: (b, c, 0, 0)),
            ],
            out_specs=pl.BlockSpec((1, CB, H, W), lambda b, c, n_ref: (b, c, 0, 0)),
        ),
        out_shape=jax.ShapeDtypeStruct((B, C, H, W), jnp.float32),
    )(n.astype(jnp.int32), g3, x)
    return out
